# Initial kernel scaffold; baseline (speedup 1.0000x reference)
import jax
import jax.numpy as jnp
from jax.experimental import pallas as pl


def kernel(pos, A, batch, edge_src, edge_dst, edge_shifts, cell, atom_emb, w_out, e0_lut):
    raise NotImplementedError("write your pallas kernel here")


# speedup vs baseline: 1.3177x; 1.3177x over previous
import jax
import jax.numpy as jnp
from jax import lax
from jax.experimental import pallas as pl
from jax.experimental.pallas import tpu as pltpu

_TC_PARAMS = pltpu.CompilerParams(vmem_limit_bytes=4 * 1024 * 1024)

BR = 8
BE = 512
BA = 128
NSPEC_MAX = 128


def _edge_body(psx, psy, psz, pdx, pdy, pdz, sx, sy, sz, bb, cell9, out):
  b = bb[...]
  c9 = cell9[...]
  s0, s1, s2 = sx[...], sy[...], sz[...]
  zeros = jnp.zeros_like(s0)
  C = [zeros] * 9
  for k in range(32):
    m = (b == k).astype(jnp.float32)
    for j in range(9):
      C[j] = C[j] + m * c9[j, k]
  sv0 = s0 * C[0] + s1 * C[3] + s2 * C[6]
  sv1 = s0 * C[1] + s1 * C[4] + s2 * C[7]
  sv2 = s0 * C[2] + s1 * C[5] + s2 * C[8]
  vx = pdx[...] - psx[...] + sv0
  vy = pdy[...] - psy[...] + sv1
  vz = pdz[...] - psz[...] + sv2
  out[...] = jnp.sqrt(vx * vx + vy * vy + vz * vz + 1e-12)


def _lut_body(emb_ref, w_ref, e0_ref, o_ref):
  o_ref[...] = e0_ref[...] + jnp.sum(
      emb_ref[...] * w_ref[...], axis=1, keepdims=True)


def _combine_body(a_ref, nbr_ref, lut_ref, o_ref):
  a = a_ref[...]
  lutv = lut_ref[...]
  acc = jnp.zeros_like(nbr_ref[...])
  for k in range(NSPEC_MAX):
    acc = acc + (a == k).astype(jnp.float32) * lutv[0, k]
  o_ref[...] = acc + 0.01 * nbr_ref[...]


def kernel(pos, A, batch, edge_src, edge_dst, edge_shifts, cell, atom_emb,
           w_out, e0_lut):
  N = pos.shape[0]
  E = edge_src.shape[0]
  S, D = atom_emb.shape
  f32 = jnp.float32

  ps = jnp.take(pos, edge_src, axis=0)
  pd = jnp.take(pos, edge_dst, axis=0)
  b_e = jnp.take(batch, edge_src, axis=0)

  RE = E // BE
  def cols2(x):
    return x.reshape(RE, BE)
  psx, psy, psz = (cols2(ps[:, i]) for i in range(3))
  pdx, pdy, pdz = (cols2(pd[:, i]) for i in range(3))
  sx, sy, sz = (cols2(edge_shifts[:, i]) for i in range(3))
  bb = cols2(b_e)
  cell9 = cell.reshape(cell.shape[0], 9).T

  eb = pl.BlockSpec((BR, BE), lambda i: (i, 0))
  r2d = pl.pallas_call(
      _edge_body,
      grid=(-(-RE // BR),),
      in_specs=[eb] * 10 + [pl.BlockSpec((9, 32), lambda i: (0, 0))],
      out_specs=eb,
      out_shape=jax.ShapeDtypeStruct((RE, BE), f32),
      compiler_params=_TC_PARAMS,
  )(psx, psy, psz, pdx, pdy, pdz, sx, sy, sz, bb, cell9)

  nbr = jax.ops.segment_sum(r2d.reshape(-1), edge_src, num_segments=N)

  SP = NSPEC_MAX
  emb_pad = jnp.pad(atom_emb, ((0, SP - S), (0, 0)))
  e0_pad = jnp.pad(e0_lut, (0, SP - S))[:, None]
  lut2 = pl.pallas_call(
      _lut_body,
      out_shape=jax.ShapeDtypeStruct((SP, 1), f32),
      compiler_params=_TC_PARAMS,
  )(emb_pad, w_out[None, :], e0_pad)
  lutrow = lut2.reshape(1, SP)

  NP = -(-N // BA) * BA
  RA = NP // BA
  A_pad = jnp.pad(A, (0, NP - N)).reshape(RA, BA)
  nbr_pad = jnp.pad(nbr, (0, NP - N)).reshape(RA, BA)
  ab = pl.BlockSpec((BR, BA), lambda i: (i, 0))
  out2 = pl.pallas_call(
      _combine_body,
      grid=(-(-RA // BR),),
      in_specs=[ab, ab, pl.BlockSpec((1, SP), lambda i: (0, 0))],
      out_specs=ab,
      out_shape=jax.ShapeDtypeStruct((RA, BA), f32),
      compiler_params=_TC_PARAMS,
  )(A_pad, nbr_pad, lutrow)
  return out2.reshape(-1)[:N]
